# baseline (device time: 17119 ns/iter reference)
import os

import jax
import jax.numpy as jnp
from jax import lax
from jax.experimental import pallas as pl
from jax.experimental.pallas import tpu as pltpu

N_DEV = 4
NC = int(os.environ.get("NC", "4"))


def kernel(x):
    m, n = x.shape
    half = m // 2
    q = m // 4
    e = m // 8
    w = n // NC

    n_xchg = 2 * (2 * NC) + 2 * NC + 2 * NC + 2 * (2 * NC)

    def body(x_ref, out, rA1, rA2, rB1, rB2, send_sems, recv_sems):
        p = lax.axis_index("i")
        x_me = p // 2
        y_me = (p % 2) ^ x_me
        py = p ^ 1
        px = 3 - p

        barrier_sem = pltpu.get_barrier_semaphore()
        if os.environ.get("BAR", "0") == "1":
            for nbr in [py, px]:
                pl.semaphore_signal(
                    barrier_sem, inc=1,
                    device_id=(nbr,), device_id_type=pl.DeviceIdType.MESH,
                )
            pl.semaphore_wait(barrier_sem, 2)
        else:
            pl.semaphore_signal(
                barrier_sem, inc=1,
                device_id=(p,), device_id_type=pl.DeviceIdType.MESH,
            )
            pl.semaphore_wait(barrier_sem, 1)

        kA1 = q * y_me
        sA1 = q * (1 - y_me)
        kA2 = kA1 + e * x_me
        sA2 = kA1 + e * (1 - x_me)
        kB1 = half + q * x_me
        sB1 = half + q * (1 - x_me)
        kB2 = kB1 + e * y_me
        sB2 = kB1 + e * (1 - y_me)
        hA0 = e * (1 - x_me)
        hA1 = e * x_me
        hB0 = e * (1 - y_me)
        hB1 = e * y_me

        _ABLATE = os.environ.get("ABLATE") == "1"

        class _Dummy:
            def start(self):
                pass

            def wait_recv(self):
                pass

            def wait_send(self):
                pass

        sem_ctr = [0]
        rdmas = []

        def xchg(src, dst, target):
            if _ABLATE:
                return _Dummy()
            i = sem_ctr[0]
            sem_ctr[0] += 1
            r = pltpu.make_async_remote_copy(
                src_ref=src, dst_ref=dst,
                send_sem=send_sems.at[i], recv_sem=recv_sems.at[i],
                device_id=(target,), device_id_type=pl.DeviceIdType.MESH,
            )
            r.start()
            rdmas.append(r)
            return r

        s1a0, s1b0, s1a1, s1b1 = [], [], [], []
        for c in range(NC):
            cs = pl.ds(c * w, w)
            out[pl.ds(sA1, q), cs] = x_ref[pl.ds(sA1, q), cs].astype(jnp.bfloat16)
            s1a0.append(xchg(out.at[pl.ds(sA1 + hA0, e), cs],
                             rA1.at[pl.ds(hA0, e), cs], py))
            out[pl.ds(sB1, q), cs] = x_ref[pl.ds(sB1, q), cs].astype(jnp.bfloat16)
            s1b0.append(xchg(out.at[pl.ds(sB1 + hB0, e), cs],
                             rB1.at[pl.ds(hB0, e), cs], px))
        for c in range(NC):
            cs = pl.ds(c * w, w)
            s1a1.append(xchg(out.at[pl.ds(sA1 + hA1, e), cs],
                             rA1.at[pl.ds(hA1, e), cs], py))
            s1b1.append(xchg(out.at[pl.ds(sB1 + hB1, e), cs],
                             rB1.at[pl.ds(hB1, e), cs], px))
        s2a, s2b = [], []
        for c in range(NC):
            cs = pl.ds(c * w, w)
            s1a0[c].wait_recv()
            out[pl.ds(sA2, e), cs] = (
                x_ref[pl.ds(sA2, e), cs].astype(jnp.bfloat16)
                + rA1[pl.ds(hA0, e), cs]
            )
            s2a.append(xchg(out.at[pl.ds(sA2, e), cs], rA2.at[:, cs], px))
            s1b0[c].wait_recv()
            out[pl.ds(sB2, e), cs] = (
                x_ref[pl.ds(sB2, e), cs].astype(jnp.bfloat16)
                + rB1[pl.ds(hB0, e), cs]
            )
            s2b.append(xchg(out.at[pl.ds(sB2, e), cs], rB2.at[:, cs], py))

        s3a, s3b = [], []
        s4ao, s4bo = [], []
        for c in range(NC):
            cs = pl.ds(c * w, w)
            s1a1[c].wait_recv()
            s2a[c].wait_recv()
            out[pl.ds(kA2, e), cs] = (
                x_ref[pl.ds(kA2, e), cs].astype(jnp.bfloat16)
                + rA1[pl.ds(hA1, e), cs]
                + rA2[:, cs]
            )
            s3a.append(xchg(out.at[pl.ds(kA2, e), cs],
                            out.at[pl.ds(kA2, e), cs], px))
            s4ao.append(xchg(out.at[pl.ds(kA2, e), cs],
                             out.at[pl.ds(kA2, e), cs], py))
            s1b1[c].wait_recv()
            s2b[c].wait_recv()
            out[pl.ds(kB2, e), cs] = (
                x_ref[pl.ds(kB2, e), cs].astype(jnp.bfloat16)
                + rB1[pl.ds(hB1, e), cs]
                + rB2[:, cs]
            )
            s3b.append(xchg(out.at[pl.ds(kB2, e), cs],
                            out.at[pl.ds(kB2, e), cs], py))
            s4bo.append(xchg(out.at[pl.ds(kB2, e), cs],
                             out.at[pl.ds(kB2, e), cs], px))

        s4ar, s4br = [], []
        for c in range(NC):
            cs = pl.ds(c * w, w)
            s3a[c].wait_recv()
            s4ar.append(xchg(out.at[pl.ds(sA2, e), cs],
                             out.at[pl.ds(sA2, e), cs], py))
            s3b[c].wait_recv()
            s4br.append(xchg(out.at[pl.ds(sB2, e), cs],
                             out.at[pl.ds(sB2, e), cs], px))

        for c in range(NC):
            s4ao[c].wait_recv()
            s4bo[c].wait_recv()
            s4ar[c].wait_recv()
            s4br[c].wait_recv()
        for r in rdmas:
            r.wait_send()

    return pl.pallas_call(
        body,
        out_shape=jax.ShapeDtypeStruct((m, n), jnp.bfloat16),
        in_specs=[pl.BlockSpec(memory_space=pltpu.VMEM)],
        out_specs=pl.BlockSpec(memory_space=pltpu.VMEM),
        scratch_shapes=[
            pltpu.VMEM((q, n), jnp.bfloat16),
            pltpu.VMEM((e, n), jnp.bfloat16),
            pltpu.VMEM((q, n), jnp.bfloat16),
            pltpu.VMEM((e, n), jnp.bfloat16),
            pltpu.SemaphoreType.DMA((n_xchg,)),
            pltpu.SemaphoreType.DMA((n_xchg,)),
        ],
        compiler_params=pltpu.CompilerParams(collective_id=0),
    )(x)


# device time: 16619 ns/iter; 1.0301x vs baseline; 1.0301x over previous
import os

import jax
import jax.numpy as jnp
from jax import lax
from jax.experimental import pallas as pl
from jax.experimental.pallas import tpu as pltpu

N_DEV = 4
NC = int(os.environ.get("NC", "4"))


def kernel(x):
    m, n = x.shape
    half = m // 2
    q = m // 4
    e = m // 8
    w = n // NC

    n_xchg = 4 * 2 * NC

    def body(x_ref, out, rA1, rA2, rB1, rB2, send_sems, recv_sems):
        p = lax.axis_index("i")
        x_me = p // 2
        y_me = (p % 2) ^ x_me
        py = p ^ 1
        px = 3 - p

        barrier_sem = pltpu.get_barrier_semaphore()
        if os.environ.get("BAR", "0") == "1":
            for nbr in [py, px]:
                pl.semaphore_signal(
                    barrier_sem, inc=1,
                    device_id=(nbr,), device_id_type=pl.DeviceIdType.MESH,
                )
            pl.semaphore_wait(barrier_sem, 2)
        else:
            pl.semaphore_signal(
                barrier_sem, inc=1,
                device_id=(p,), device_id_type=pl.DeviceIdType.MESH,
            )
            pl.semaphore_wait(barrier_sem, 1)

        kA1 = q * y_me
        sA1 = q * (1 - y_me)
        kA2 = kA1 + e * x_me
        sA2 = kA1 + e * (1 - x_me)
        kB1 = half + q * x_me
        sB1 = half + q * (1 - x_me)
        kB2 = kB1 + e * y_me
        sB2 = kB1 + e * (1 - y_me)
        hA0 = e * (1 - x_me)
        hA1 = e * x_me
        hB0 = e * (1 - y_me)
        hB1 = e * y_me

        _ABLATE = os.environ.get("ABLATE") == "1"

        class _Dummy:
            def start(self):
                pass

            def wait_recv(self):
                pass

            def wait_send(self):
                pass

        sem_ctr = [0]
        rdmas = []

        def xchg(src, dst, target):
            if _ABLATE:
                return _Dummy()
            i = sem_ctr[0]
            sem_ctr[0] += 1
            r = pltpu.make_async_remote_copy(
                src_ref=src, dst_ref=dst,
                send_sem=send_sems.at[i], recv_sem=recv_sems.at[i],
                device_id=(target,), device_id_type=pl.DeviceIdType.MESH,
            )
            r.start()
            rdmas.append(r)
            return r

        s1a, s1b = [], []
        for c in range(NC):
            cs = pl.ds(c * w, w)
            out[pl.ds(sA1, q), cs] = x_ref[pl.ds(sA1, q), cs].astype(jnp.bfloat16)
            s1a.append(xchg(out.at[pl.ds(sA1, q), cs], rA1.at[:, cs], py))
            out[pl.ds(sB1, q), cs] = x_ref[pl.ds(sB1, q), cs].astype(jnp.bfloat16)
            s1b.append(xchg(out.at[pl.ds(sB1, q), cs], rB1.at[:, cs], px))

        s2a, s2b = [], []
        for c in range(NC):
            cs = pl.ds(c * w, w)
            s1a[c].wait_recv()
            out[pl.ds(sA2, e), cs] = (
                x_ref[pl.ds(sA2, e), cs].astype(jnp.bfloat16)
                + rA1[pl.ds(hA0, e), cs]
            )
            s2a.append(xchg(out.at[pl.ds(sA2, e), cs], rA2.at[:, cs], px))
            s1b[c].wait_recv()
            out[pl.ds(sB2, e), cs] = (
                x_ref[pl.ds(sB2, e), cs].astype(jnp.bfloat16)
                + rB1[pl.ds(hB0, e), cs]
            )
            s2b.append(xchg(out.at[pl.ds(sB2, e), cs], rB2.at[:, cs], py))

        s3a, s3b = [], []
        for c in range(NC):
            cs = pl.ds(c * w, w)
            s2a[c].wait_recv()
            out[pl.ds(kA2, e), cs] = (
                x_ref[pl.ds(kA2, e), cs].astype(jnp.bfloat16)
                + rA1[pl.ds(hA1, e), cs]
                + rA2[:, cs]
            )
            s3a.append(xchg(out.at[pl.ds(kA2, e), cs],
                            out.at[pl.ds(kA2, e), cs], px))
            s2b[c].wait_recv()
            out[pl.ds(kB2, e), cs] = (
                x_ref[pl.ds(kB2, e), cs].astype(jnp.bfloat16)
                + rB1[pl.ds(hB1, e), cs]
                + rB2[:, cs]
            )
            s3b.append(xchg(out.at[pl.ds(kB2, e), cs],
                            out.at[pl.ds(kB2, e), cs], py))

        s4a, s4b = [], []
        for c in range(NC):
            cs = pl.ds(c * w, w)
            s3a[c].wait_recv()
            s4a.append(xchg(out.at[pl.ds(kA1, q), cs],
                            out.at[pl.ds(kA1, q), cs], py))
            s3b[c].wait_recv()
            s4b.append(xchg(out.at[pl.ds(kB1, q), cs],
                            out.at[pl.ds(kB1, q), cs], px))

        for c in range(NC):
            s4a[c].wait_recv()
            s4b[c].wait_recv()
        for r in rdmas:
            r.wait_send()

    return pl.pallas_call(
        body,
        out_shape=jax.ShapeDtypeStruct((m, n), jnp.bfloat16),
        in_specs=[pl.BlockSpec(memory_space=pltpu.VMEM)],
        out_specs=pl.BlockSpec(memory_space=pltpu.VMEM),
        scratch_shapes=[
            pltpu.VMEM((q, n), jnp.bfloat16),
            pltpu.VMEM((e, n), jnp.bfloat16),
            pltpu.VMEM((q, n), jnp.bfloat16),
            pltpu.VMEM((e, n), jnp.bfloat16),
            pltpu.SemaphoreType.DMA((n_xchg,)),
            pltpu.SemaphoreType.DMA((n_xchg,)),
        ],
        compiler_params=pltpu.CompilerParams(collective_id=0),
    )(x)
